# baseline (device time: 215579 ns/iter reference)
import jax
import jax.numpy as jnp
from jax import lax
from jax.experimental import pallas as pl
from jax.experimental.pallas import tpu as pltpu

N_Y = 4
V_PER = 8192


def kernel(ids, E):
    my_y = lax.axis_index("y")

    local = ids - my_y * V_PER
    in_range = (local >= 0) & (local < V_PER)
    safe = jnp.clip(local, 0, V_PER - 1)
    partial = jnp.where(in_range[:, None], jnp.take(E, safe, axis=0), 0.0)
    partial = partial.astype(jnp.float32)

    t, d = partial.shape

    def body(p_ref, out_ref, comm_ref, send_sems, recv_sems):
        my_x = lax.axis_index("x")
        yy = lax.axis_index("y")
        my_z = lax.axis_index("z")
        right = (yy + 1) % N_Y
        left = (yy - 1) % N_Y

        barrier_sem = pltpu.get_barrier_semaphore()
        for nbr in (left, right):
            pl.semaphore_signal(
                barrier_sem,
                inc=1,
                device_id=(my_x, nbr, my_z),
                device_id_type=pl.DeviceIdType.MESH,
            )
        pl.semaphore_wait(barrier_sem, 2)

        out_ref[:, :] = p_ref[:, :]
        comm_ref[0, :, :] = p_ref[:, :]

        for h in range(N_Y - 1):
            rdma = pltpu.make_async_remote_copy(
                src_ref=comm_ref.at[h],
                dst_ref=comm_ref.at[h + 1],
                send_sem=send_sems.at[h],
                recv_sem=recv_sems.at[h],
                device_id=(my_x, right, my_z),
                device_id_type=pl.DeviceIdType.MESH,
            )
            rdma.start()
            rdma.wait()
            out_ref[:, :] = out_ref[:, :] + comm_ref[h + 1, :, :]

    return pl.pallas_call(
        body,
        out_shape=jax.ShapeDtypeStruct((t, d), jnp.float32),
        in_specs=[pl.BlockSpec(memory_space=pltpu.VMEM)],
        out_specs=pl.BlockSpec(memory_space=pltpu.VMEM),
        scratch_shapes=[
            pltpu.VMEM((N_Y, t, d), jnp.float32),
            pltpu.SemaphoreType.DMA((N_Y - 1,)),
            pltpu.SemaphoreType.DMA((N_Y - 1,)),
        ],
        compiler_params=pltpu.CompilerParams(collective_id=0),
    )(partial)


# device time: 127830 ns/iter; 1.6865x vs baseline; 1.6865x over previous
import jax
import jax.numpy as jnp
from jax import lax
from jax.experimental import pallas as pl
from jax.experimental.pallas import tpu as pltpu

N_Y = 4
V_PER = 8192
HALF = 512
C = 128


def kernel(ids, E):
    my_y = lax.axis_index("y")

    local = ids - my_y * V_PER
    in_range = (local >= 0) & (local < V_PER)
    safe = jnp.clip(local, 0, V_PER - 1)
    partial = jnp.where(in_range[:, None], jnp.take(E, safe, axis=0), 0.0)
    partial = partial.astype(jnp.float32)

    t, d = partial.shape

    def body(p_ref, out_ref, acc_ref, rbuf,
             rs_send, rs_recv, ag_send, ag_recv, x_send, x_recv):
        my_x = lax.axis_index("x")
        yy = lax.axis_index("y")
        my_z = lax.axis_index("z")
        right = (yy + 1) % N_Y
        left = (yy - 1) % N_Y
        row0 = my_x * HALF

        barrier_sem = pltpu.get_barrier_semaphore()
        for nbr in ((my_x, left, my_z), (my_x, right, my_z),
                    (1 - my_x, yy, my_z)):
            pl.semaphore_signal(
                barrier_sem, inc=1,
                device_id=nbr, device_id_type=pl.DeviceIdType.MESH,
            )
        pl.semaphore_wait(barrier_sem, 3)

        acc_ref[:, :] = p_ref[pl.ds(row0, HALF), :]

        x_rdmas = []

        def x_forward(chunk, k):
            rd = pltpu.make_async_remote_copy(
                src_ref=out_ref.at[pl.ds(row0 + chunk * C, C), :],
                dst_ref=out_ref.at[pl.ds(row0 + chunk * C, C), :],
                send_sem=x_send.at[k],
                recv_sem=x_recv.at[k],
                device_id=(1 - my_x, yy, my_z),
                device_id_type=pl.DeviceIdType.MESH,
            )
            rd.start()
            x_rdmas.append(rd)

        for p in range(N_Y - 1):
            s = (yy - p) % N_Y
            r = (yy - p - 1) % N_Y
            rd = pltpu.make_async_remote_copy(
                src_ref=acc_ref.at[pl.ds(s * C, C), :],
                dst_ref=rbuf.at[p],
                send_sem=rs_send.at[p],
                recv_sem=rs_recv.at[p],
                device_id=(my_x, right, my_z),
                device_id_type=pl.DeviceIdType.MESH,
            )
            rd.start()
            rd.wait()
            acc_ref[pl.ds(r * C, C), :] = (
                acc_ref[pl.ds(r * C, C), :] + rbuf[p]
            )

        own = (yy + 1) % N_Y
        out_ref[pl.ds(row0 + own * C, C), :] = acc_ref[pl.ds(own * C, C), :]
        x_forward(own, 0)

        for p in range(N_Y - 1):
            a = (own - p) % N_Y
            g = (yy - p) % N_Y
            rd = pltpu.make_async_remote_copy(
                src_ref=out_ref.at[pl.ds(row0 + a * C, C), :],
                dst_ref=out_ref.at[pl.ds(row0 + a * C, C), :],
                send_sem=ag_send.at[p],
                recv_sem=ag_recv.at[p],
                device_id=(my_x, right, my_z),
                device_id_type=pl.DeviceIdType.MESH,
            )
            rd.start()
            rd.wait()
            x_forward(g, p + 1)

        for rd in x_rdmas:
            rd.wait()

    return pl.pallas_call(
        body,
        out_shape=jax.ShapeDtypeStruct((t, d), jnp.float32),
        in_specs=[pl.BlockSpec(memory_space=pltpu.VMEM)],
        out_specs=pl.BlockSpec(memory_space=pltpu.VMEM),
        scratch_shapes=[
            pltpu.VMEM((HALF, d), jnp.float32),
            pltpu.VMEM((N_Y - 1, C, d), jnp.float32),
            pltpu.SemaphoreType.DMA((N_Y - 1,)),
            pltpu.SemaphoreType.DMA((N_Y - 1,)),
            pltpu.SemaphoreType.DMA((N_Y - 1,)),
            pltpu.SemaphoreType.DMA((N_Y - 1,)),
            pltpu.SemaphoreType.DMA((N_Y,)),
            pltpu.SemaphoreType.DMA((N_Y,)),
        ],
        compiler_params=pltpu.CompilerParams(collective_id=0),
    )(partial)


# device time: 69234 ns/iter; 3.1138x vs baseline; 1.8463x over previous
import jax
import jax.numpy as jnp
from jax import lax
from jax.experimental import pallas as pl
from jax.experimental.pallas import tpu as pltpu

N_Y = 4
V_PER = 8192
HALF = 512
C = 128


def kernel(ids, E):
    my_y = lax.axis_index("y")
    my_x = lax.axis_index("x")

    my_ids = lax.dynamic_slice(ids, (my_x * HALF,), (HALF,))
    local = my_ids - my_y * V_PER
    in_range = (local >= 0) & (local < V_PER)
    safe = jnp.clip(local, 0, V_PER - 1)
    partial = jnp.where(in_range[:, None], jnp.take(E, safe, axis=0), 0.0)
    partial = partial.astype(jnp.bfloat16)

    t, d = ids.shape[0], E.shape[1]

    def body(p_ref, out_ref, acc_ref, gbuf, xbuf, rbuf,
             rs_send, rs_recv, ag_send, ag_recv, x_send, x_recv):
        xx = lax.axis_index("x")
        yy = lax.axis_index("y")
        zz = lax.axis_index("z")
        right = (yy + 1) % N_Y
        left = (yy - 1) % N_Y

        barrier_sem = pltpu.get_barrier_semaphore()
        for nbr in ((xx, left, zz), (xx, right, zz), (1 - xx, yy, zz)):
            pl.semaphore_signal(
                barrier_sem, inc=1,
                device_id=nbr, device_id_type=pl.DeviceIdType.MESH,
            )
        pl.semaphore_wait(barrier_sem, 3)

        acc_ref[:, :] = p_ref[:, :]

        x_rdmas = []

        def x_forward(chunk, k):
            rd = pltpu.make_async_remote_copy(
                src_ref=gbuf.at[pl.ds(chunk * C, C), :],
                dst_ref=xbuf.at[pl.ds(chunk * C, C), :],
                send_sem=x_send.at[k],
                recv_sem=x_recv.at[k],
                device_id=(1 - xx, yy, zz),
                device_id_type=pl.DeviceIdType.MESH,
            )
            rd.start()
            x_rdmas.append(rd)

        for p in range(N_Y - 1):
            s = (yy - p) % N_Y
            r = (yy - p - 1) % N_Y
            rd = pltpu.make_async_remote_copy(
                src_ref=acc_ref.at[pl.ds(s * C, C), :],
                dst_ref=rbuf.at[p],
                send_sem=rs_send.at[p],
                recv_sem=rs_recv.at[p],
                device_id=(xx, right, zz),
                device_id_type=pl.DeviceIdType.MESH,
            )
            rd.start()
            rd.wait()
            acc_ref[pl.ds(r * C, C), :] = (
                acc_ref[pl.ds(r * C, C), :] + rbuf[p]
            )

        own = (yy + 1) % N_Y
        gbuf[pl.ds(own * C, C), :] = acc_ref[pl.ds(own * C, C), :]
        x_forward(own, 0)

        for p in range(N_Y - 1):
            a = (own - p) % N_Y
            g = (yy - p) % N_Y
            rd = pltpu.make_async_remote_copy(
                src_ref=gbuf.at[pl.ds(a * C, C), :],
                dst_ref=gbuf.at[pl.ds(a * C, C), :],
                send_sem=ag_send.at[p],
                recv_sem=ag_recv.at[p],
                device_id=(xx, right, zz),
                device_id_type=pl.DeviceIdType.MESH,
            )
            rd.start()
            rd.wait()
            x_forward(g, p + 1)

        for rd in x_rdmas:
            rd.wait()
        mine0 = xx * HALF
        theirs0 = (1 - xx) * HALF
        out_ref[pl.ds(mine0, HALF), :] = gbuf[:, :].astype(jnp.float32)
        out_ref[pl.ds(theirs0, HALF), :] = xbuf[:, :].astype(jnp.float32)

    return pl.pallas_call(
        body,
        out_shape=jax.ShapeDtypeStruct((t, d), jnp.float32),
        in_specs=[pl.BlockSpec(memory_space=pltpu.VMEM)],
        out_specs=pl.BlockSpec(memory_space=pltpu.VMEM),
        scratch_shapes=[
            pltpu.VMEM((HALF, d), jnp.bfloat16),
            pltpu.VMEM((HALF, d), jnp.bfloat16),
            pltpu.VMEM((HALF, d), jnp.bfloat16),
            pltpu.VMEM((N_Y - 1, C, d), jnp.bfloat16),
            pltpu.SemaphoreType.DMA((N_Y - 1,)),
            pltpu.SemaphoreType.DMA((N_Y - 1,)),
            pltpu.SemaphoreType.DMA((N_Y - 1,)),
            pltpu.SemaphoreType.DMA((N_Y - 1,)),
            pltpu.SemaphoreType.DMA((N_Y,)),
            pltpu.SemaphoreType.DMA((N_Y,)),
        ],
        compiler_params=pltpu.CompilerParams(collective_id=0),
    )(partial)


# device time: 53589 ns/iter; 4.0228x vs baseline; 1.2919x over previous
import jax
import jax.numpy as jnp
from jax import lax
from jax.experimental import pallas as pl
from jax.experimental.pallas import tpu as pltpu

N_Y = 4
V_PER = 8192
HALF = 512
C = 128
K = 32


def kernel(ids, E):
    my_y = lax.axis_index("y")
    my_x = lax.axis_index("x")

    my_ids = lax.dynamic_slice(ids, (my_x * HALF,), (HALF,))
    local = my_ids - my_y * V_PER
    in_range = (local >= 0) & (local < V_PER)
    safe = jnp.clip(local, 0, V_PER - 1).astype(jnp.int32)
    scale = in_range.astype(jnp.float32)[:, None]

    t, d = ids.shape[0], E.shape[1]

    def body(ids_ref, scale_ref, e_ref, out_ref,
             fbuf, acc_ref, gbuf, xbuf, rbuf, gsem,
             rs_send, rs_recv, ag_send, ag_recv, x_send, x_recv):
        xx = lax.axis_index("x")
        yy = lax.axis_index("y")
        zz = lax.axis_index("z")
        right = (yy + 1) % N_Y
        left = (yy - 1) % N_Y

        barrier_sem = pltpu.get_barrier_semaphore()
        for nbr in ((xx, left, zz), (xx, right, zz), (1 - xx, yy, zz)):
            pl.semaphore_signal(
                barrier_sem, inc=1,
                device_id=nbr, device_id_type=pl.DeviceIdType.MESH,
            )
        pl.semaphore_wait(barrier_sem, 3)

        def row_dma(row):
            return pltpu.make_async_copy(
                e_ref.at[pl.ds(ids_ref[row], 1), :],
                fbuf.at[pl.ds(row, 1), :],
                gsem.at[row % K],
            )

        def gather_chunk(j):
            base = ((yy - j) % N_Y) * C

            def gbody(i, _):
                row_dma(base + i).start()

                @pl.when(i >= K)
                def _():
                    row_dma(base + i - K).wait()

                return 0

            lax.fori_loop(0, C, gbody, 0, unroll=4)
            for jj in range(K):
                row_dma(base + C - K + jj).wait()
            acc_ref[pl.ds(base, C), :] = (
                fbuf[pl.ds(base, C), :] * scale_ref[pl.ds(base, C), :]
            ).astype(jnp.bfloat16)

        x_rdmas = []

        def x_forward(chunk, k):
            rd = pltpu.make_async_remote_copy(
                src_ref=gbuf.at[pl.ds(chunk * C, C), :],
                dst_ref=xbuf.at[pl.ds(chunk * C, C), :],
                send_sem=x_send.at[k],
                recv_sem=x_recv.at[k],
                device_id=(1 - xx, yy, zz),
                device_id_type=pl.DeviceIdType.MESH,
            )
            rd.start()
            x_rdmas.append(rd)

        gather_chunk(0)
        for p in range(N_Y - 1):
            s = (yy - p) % N_Y
            r = (yy - p - 1) % N_Y
            rd = pltpu.make_async_remote_copy(
                src_ref=acc_ref.at[pl.ds(s * C, C), :],
                dst_ref=rbuf.at[p],
                send_sem=rs_send.at[p],
                recv_sem=rs_recv.at[p],
                device_id=(xx, right, zz),
                device_id_type=pl.DeviceIdType.MESH,
            )
            rd.start()
            gather_chunk(p + 1)
            rd.wait()
            acc_ref[pl.ds(r * C, C), :] = (
                acc_ref[pl.ds(r * C, C), :] + rbuf[p]
            )

        own = (yy + 1) % N_Y
        gbuf[pl.ds(own * C, C), :] = acc_ref[pl.ds(own * C, C), :]
        x_forward(own, 0)

        for p in range(N_Y - 1):
            a = (own - p) % N_Y
            g = (yy - p) % N_Y
            rd = pltpu.make_async_remote_copy(
                src_ref=gbuf.at[pl.ds(a * C, C), :],
                dst_ref=gbuf.at[pl.ds(a * C, C), :],
                send_sem=ag_send.at[p],
                recv_sem=ag_recv.at[p],
                device_id=(xx, right, zz),
                device_id_type=pl.DeviceIdType.MESH,
            )
            rd.start()
            rd.wait()
            x_forward(g, p + 1)

        for rd in x_rdmas:
            rd.wait()
        mine0 = xx * HALF
        theirs0 = (1 - xx) * HALF
        out_ref[pl.ds(mine0, HALF), :] = gbuf[:, :].astype(jnp.float32)
        out_ref[pl.ds(theirs0, HALF), :] = xbuf[:, :].astype(jnp.float32)

    return pl.pallas_call(
        body,
        out_shape=jax.ShapeDtypeStruct((t, d), jnp.float32),
        in_specs=[
            pl.BlockSpec(memory_space=pltpu.SMEM),
            pl.BlockSpec(memory_space=pltpu.VMEM),
            pl.BlockSpec(memory_space=pl.ANY),
        ],
        out_specs=pl.BlockSpec(memory_space=pltpu.VMEM),
        scratch_shapes=[
            pltpu.VMEM((HALF, d), jnp.float32),
            pltpu.VMEM((HALF, d), jnp.bfloat16),
            pltpu.VMEM((HALF, d), jnp.bfloat16),
            pltpu.VMEM((HALF, d), jnp.bfloat16),
            pltpu.VMEM((N_Y - 1, C, d), jnp.bfloat16),
            pltpu.SemaphoreType.DMA((K,)),
            pltpu.SemaphoreType.DMA((N_Y - 1,)),
            pltpu.SemaphoreType.DMA((N_Y - 1,)),
            pltpu.SemaphoreType.DMA((N_Y - 1,)),
            pltpu.SemaphoreType.DMA((N_Y - 1,)),
            pltpu.SemaphoreType.DMA((N_Y,)),
            pltpu.SemaphoreType.DMA((N_Y,)),
        ],
        compiler_params=pltpu.CompilerParams(collective_id=0),
    )(safe, scale, E)
